# baseline (speedup 1.0000x reference)
import jax
import jax.numpy as jnp
from jax import lax
from jax.experimental import pallas as pl
from jax.experimental.pallas import tpu as pltpu
from jax.experimental.pallas import tpu_sc as plsc

_B = 16384
_V = 100000
_NDIS = 26
_NCONT = 13
_D = 16
_F1 = 13

_NCORES = 2
_NSUB = 16
_NW = _NCORES * _NSUB
_BPW = _B // _NW
_NC = 64
_NCHUNK = _BPW // _NC
_NGRP = _NC // 16


def _stage_x(xtr, xt_v, wid):
    pltpu.sync_copy(xtr.at[wid], xt_v)


def _fm_partial(xt_v, idx3_v, rows_v, wfm, sem_fm, nf, fbase, s_fn, q_fn):
    @pl.loop(0, _NCHUNK)
    def _chunk(c):
        cbase = c * _NC

        @pl.loop(0, _NGRP)
        def _i3(g):
            off = pl.multiple_of(cbase + g * 16, 16)
            for f in range(nf):
                xv = xt_v[pl.ds((fbase + f) * _BPW + off, 16)]
                bx = xv.astype(jnp.int32)
                for d in range(_D):
                    idx3_v[pl.ds((f * _D + d) * _NC + g * 16, 16)] = (
                        bx + (f * _D + d) * _V)

        cp_fm = pltpu.async_copy(wfm.at[idx3_v], rows_v, sem_fm)
        cp_fm.wait()

        @pl.loop(0, _NGRP)
        def _fm(g):
            off = pl.multiple_of(cbase + g * 16, 16)
            goff = g * 16
            s_out = []
            cq = jnp.zeros((16,), jnp.float32)
            for h in range(2):
                d0 = h * 8
                s_acc = [jnp.zeros((16,), jnp.float32) for _ in range(8)]
                q_acc = [jnp.zeros((16,), jnp.float32) for _ in range(8)]
                for f in range(nf):
                    xfv = xt_v[pl.ds((fbase + f) * _BPW + off, 16)]
                    for dd in range(8):
                        rv = rows_v[
                            pl.ds((f * _D + d0 + dd) * _NC + goff, 16)]
                        t = rv * xfv
                        s_acc[dd] = s_acc[dd] + t
                        q_acc[dd] = q_acc[dd] + t * t
                for dd in range(8):
                    cq = cq + q_acc[dd]
                s_out.extend(s_acc)
            q_fn(off, cq)
            s_fn(off, s_out)


def _body1(xtr, wfm, wlin, s1, scal1, xt_v, idx2_v, idx3_v, rows_v, lin_v,
           s1_v, scal_v, sem_fm, sem_lin):
    wid = lax.axis_index("s") * _NCORES + lax.axis_index("c")
    _stage_x(xtr, xt_v, wid)

    @pl.loop(0, _BPW // 16)
    def _idx(g):
        off = pl.multiple_of(g * 16, 16)
        for f in range(_NDIS):
            xv = xt_v[pl.ds(f * _BPW + off, 16)]
            idx2_v[pl.ds(f * _BPW + off, 16)] = xv.astype(jnp.int32) + f * _V

    cp_lin = pltpu.async_copy(wlin.at[idx2_v], lin_v, sem_lin)
    cp_lin.wait()

    def s_fn(off, s_out):
        for d in range(_D):
            s1_v[pl.ds(d * _BPW + off, 16)] = s_out[d]

    def q_fn(off, cq):
        linacc = lin_v[pl.ds(off, 16)]
        for f in range(1, _NDIS):
            linacc = linacc + lin_v[pl.ds(f * _BPW + off, 16)]
        scal_v[pl.ds(off, 16)] = linacc - 0.5 * cq

    _fm_partial(xt_v, idx3_v, rows_v, wfm, sem_fm, _F1, 0, s_fn, q_fn)

    pltpu.sync_copy(s1_v, s1.at[wid])
    base = pl.multiple_of(wid * _BPW, 8)
    pltpu.sync_copy(scal_v, scal1.at[pl.ds(base, _BPW)])


def _body2(xtr, wfm, wc, s1, scal1, out, xt_v, idx3_v, rows_v, wc_v, wcb_v,
           s1_v, scal_v, out_v, sem_fm):
    wid = lax.axis_index("s") * _NCORES + lax.axis_index("c")
    _stage_x(xtr, xt_v, wid)
    pltpu.sync_copy(wc, wc_v)
    pltpu.sync_copy(s1.at[wid], s1_v)
    base = pl.multiple_of(wid * _BPW, 8)
    pltpu.sync_copy(scal1.at[pl.ds(base, _BPW)], scal_v)

    zeros = jnp.zeros((16,), jnp.float32)
    for cf in range(_NCONT):
        row = wc_v[pl.ds(cf * _D, _D)]
        for d in range(_D):
            wcb_v[pl.ds((cf * _D + d) * 16, 16)] = zeros + row[d]

    def s_fn(off, s_out):
        css = jnp.zeros((16,), jnp.float32)
        cq2 = jnp.zeros((16,), jnp.float32)
        lincont = jnp.zeros((16,), jnp.float32)
        xcs = []
        for cf in range(_NCONT):
            xcv = xt_v[pl.ds((_NDIS + cf) * _BPW + off, 16)]
            xcs.append(xcv)
            lincont = lincont + xcv
        for d in range(_D):
            sd = s_out[d] + s1_v[pl.ds(d * _BPW + off, 16)]
            for cf in range(_NCONT):
                t = xcs[cf] * wcb_v[pl.ds((cf * _D + d) * 16, 16)]
                sd = sd + t
                cq2 = cq2 + t * t
            css = css + sd * sd
        z = scal_v[pl.ds(off, 16)] + lincont + 0.5 * (css - cq2)
        out_v[pl.ds(off, 16)] = 1.0 / (1.0 + jnp.exp(-z))

    def q_fn(off, cq):
        scal_v[pl.ds(off, 16)] = scal_v[pl.ds(off, 16)] - 0.5 * cq

    _fm_partial(xt_v, idx3_v, rows_v, wfm, sem_fm, _NDIS - _F1, _F1,
                s_fn, q_fn)

    pltpu.sync_copy(out_v, out.at[pl.ds(base, _BPW)])


_sds = jax.ShapeDtypeStruct
_mesh = plsc.VectorSubcoreMesh(
    core_axis_name="c", subcore_axis_name="s",
    num_cores=_NCORES, num_subcores=_NSUB)
_params = pltpu.CompilerParams(
    needs_layout_passes=False, use_tc_tiling_on_sc=False)

_call1 = pl.kernel(
    _body1,
    out_type=(_sds((_NW, _D * _BPW), jnp.float32),
              _sds((_B,), jnp.float32)),
    mesh=_mesh,
    scratch_types=[
        pltpu.VMEM(((_NDIS + _NCONT) * _BPW,), jnp.float32),
        pltpu.VMEM((_NDIS * _BPW,), jnp.int32),
        pltpu.VMEM((_F1 * _D * _NC,), jnp.int32),
        pltpu.VMEM((_F1 * _D * _NC,), jnp.float32),
        pltpu.VMEM((_NDIS * _BPW,), jnp.float32),
        pltpu.VMEM((_D * _BPW,), jnp.float32),
        pltpu.VMEM((_BPW,), jnp.float32),
        pltpu.SemaphoreType.DMA,
        pltpu.SemaphoreType.DMA,
    ],
    compiler_params=_params,
)

_call2 = pl.kernel(
    _body2,
    out_type=_sds((_B,), jnp.float32),
    mesh=_mesh,
    scratch_types=[
        pltpu.VMEM(((_NDIS + _NCONT) * _BPW,), jnp.float32),
        pltpu.VMEM(((_NDIS - _F1) * _D * _NC,), jnp.int32),
        pltpu.VMEM(((_NDIS - _F1) * _D * _NC,), jnp.float32),
        pltpu.VMEM((_NCONT * _D,), jnp.float32),
        pltpu.VMEM((_NCONT * _D * 16,), jnp.float32),
        pltpu.VMEM((_D * _BPW,), jnp.float32),
        pltpu.VMEM((_BPW,), jnp.float32),
        pltpu.VMEM((_BPW,), jnp.float32),
        pltpu.SemaphoreType.DMA,
    ],
    compiler_params=_params,
)


def kernel(X, W_lin, W_fm, W_cont):
    xtr = X.T.reshape(_NDIS + _NCONT, _NW, _BPW).transpose(1, 0, 2)
    xtr = xtr.reshape(_NW, (_NDIS + _NCONT) * _BPW)
    wfm1 = W_fm[:_F1].transpose(0, 2, 1).reshape(_F1 * _D * _V)
    wfm2 = W_fm[_F1:].transpose(0, 2, 1).reshape((_NDIS - _F1) * _D * _V)
    s1, scal1 = _call1(xtr, wfm1, W_lin.reshape(_NDIS * _V))
    out = _call2(xtr, wfm2, W_cont.reshape(_NCONT * _D), s1, scal1)
    return out.reshape(_B, 1)


# speedup vs baseline: 1.0503x; 1.0503x over previous
import jax
import jax.numpy as jnp
from jax import lax
from jax.experimental import pallas as pl
from jax.experimental.pallas import tpu as pltpu
from jax.experimental.pallas import tpu_sc as plsc

_B = 16384
_V = 100000
_NDIS = 26
_NCONT = 13
_D = 16
_NF = _NDIS + _NCONT

_NCORES = 2
_NSUB = 16
_NW = _NCORES * _NSUB
_BPW = _B // _NW
_NC = 32
_NCHUNK = _BPW // _NC
_NGRP = _NC // 16

_FMW = _NDIS * _D
_ISZ = (_FMW + _NDIS) * _NC
_RSZ = _FMW * _NC
_LSZ = _NDIS * _NC
_XSZ = _NF * _NC


def _fm_body(xtr, wfm, wlin, wc, out, xt_v, idx_v, rows_v, lin_v, wc_v,
             wcb_v, out_v, sem_a, sem_b):
    wid = lax.axis_index("s") * _NCORES + lax.axis_index("c")
    pltpu.sync_copy(wc, wc_v)

    zeros = jnp.zeros((16,), jnp.float32)
    for cf in range(_NCONT):
        row = wc_v[pl.ds(cf * _D, _D)]
        for d in range(_D):
            wcb_v[pl.ds((cf * _D + d) * 16, 16)] = zeros + row[d]

    def _stage_and_fire(c, par, sem):
        xoff = par * _XSZ
        ioff = par * _ISZ
        pltpu.sync_copy(xtr.at[wid, c], xt_v.at[pl.ds(xoff, _XSZ)])

        @pl.loop(0, _NGRP)
        def _i3(g):
            goff = g * 16
            for f in range(_NDIS):
                xv = xt_v[pl.ds(xoff + f * _NC + goff, 16)]
                bx = xv.astype(jnp.int32)
                for d in range(_D):
                    idx_v[pl.ds(ioff + (f * _D + d) * _NC + goff, 16)] = (
                        bx + (f * _D + d) * _V)
                idx_v[pl.ds(ioff + _FMW * _NC + f * _NC + goff, 16)] = (
                    bx + f * _V)

        cp_fm = pltpu.async_copy(
            wfm.at[idx_v.at[pl.ds(ioff, _RSZ)]],
            rows_v.at[pl.ds(par * _RSZ, _RSZ)], sem)
        cp_lin = pltpu.async_copy(
            wlin.at[idx_v.at[pl.ds(ioff + _RSZ, _LSZ)]],
            lin_v.at[pl.ds(par * _LSZ, _LSZ)], sem)
        return cp_fm, cp_lin

    def _compute(c, par):
        xoff = par * _XSZ
        roff = par * _RSZ
        loff = par * _LSZ

        @pl.loop(0, _NGRP)
        def _fm(g):
            goff = g * 16
            off = c * _NC + goff

            linacc = lin_v[pl.ds(loff + goff, 16)]
            for f in range(1, _NDIS):
                linacc = linacc + lin_v[pl.ds(loff + f * _NC + goff, 16)]
            for cf in range(_NCONT):
                linacc = linacc + xt_v[
                    pl.ds(xoff + (_NDIS + cf) * _NC + goff, 16)]

            css = jnp.zeros((16,), jnp.float32)
            cq = jnp.zeros((16,), jnp.float32)
            for h in range(2):
                d0 = h * 8
                s_acc = [jnp.zeros((16,), jnp.float32) for _ in range(8)]
                q_acc = [jnp.zeros((16,), jnp.float32) for _ in range(8)]
                for f in range(_NDIS):
                    xfv = xt_v[pl.ds(xoff + f * _NC + goff, 16)]
                    for dd in range(8):
                        rv = rows_v[pl.ds(
                            roff + (f * _D + d0 + dd) * _NC + goff, 16)]
                        t = rv * xfv
                        s_acc[dd] = s_acc[dd] + t
                        q_acc[dd] = q_acc[dd] + t * t
                for cf in range(_NCONT):
                    xcv = xt_v[pl.ds(xoff + (_NDIS + cf) * _NC + goff, 16)]
                    for dd in range(8):
                        t = xcv * wcb_v[pl.ds((cf * _D + d0 + dd) * 16, 16)]
                        s_acc[dd] = s_acc[dd] + t
                        q_acc[dd] = q_acc[dd] + t * t
                for dd in range(8):
                    css = css + s_acc[dd] * s_acc[dd]
                    cq = cq + q_acc[dd]

            z = linacc + 0.5 * (css - cq)
            out_v[pl.ds(off, 16)] = 1.0 / (1.0 + jnp.exp(-z))

    @pl.loop(0, _NCHUNK // 2)
    def _pair(p):
        c0 = p * 2
        cpa = _stage_and_fire(c0, 0, sem_a)
        cpb = _stage_and_fire(c0 + 1, 1, sem_b)
        cpa[0].wait()
        cpa[1].wait()
        _compute(c0, 0)
        cpb[0].wait()
        cpb[1].wait()
        _compute(c0 + 1, 1)

    base = pl.multiple_of(wid * _BPW, 8)
    pltpu.sync_copy(out_v, out.at[pl.ds(base, _BPW)])


_fm_call = pl.kernel(
    _fm_body,
    out_type=jax.ShapeDtypeStruct((_B,), jnp.float32),
    mesh=plsc.VectorSubcoreMesh(
        core_axis_name="c", subcore_axis_name="s",
        num_cores=_NCORES, num_subcores=_NSUB),
    scratch_types=[
        pltpu.VMEM((2 * _XSZ,), jnp.float32),
        pltpu.VMEM((2 * _ISZ,), jnp.int32),
        pltpu.VMEM((2 * _RSZ,), jnp.float32),
        pltpu.VMEM((2 * _LSZ,), jnp.float32),
        pltpu.VMEM((_NCONT * _D,), jnp.float32),
        pltpu.VMEM((_NCONT * _D * 16,), jnp.float32),
        pltpu.VMEM((_BPW,), jnp.float32),
        pltpu.SemaphoreType.DMA,
        pltpu.SemaphoreType.DMA,
    ],
    compiler_params=pltpu.CompilerParams(
        needs_layout_passes=False, use_tc_tiling_on_sc=False),
)


def kernel(X, W_lin, W_fm, W_cont):
    xtr = X.T.reshape(_NF, _NW, _NCHUNK, _NC).transpose(1, 2, 0, 3)
    xtr = xtr.reshape(_NW, _NCHUNK, _NF * _NC)
    wfm_flat = W_fm.transpose(0, 2, 1).reshape(_NDIS * _D * _V)
    wlin_flat = W_lin.transpose(0, 2, 1).reshape(_NDIS * _V)
    out = _fm_call(xtr, wfm_flat, wlin_flat, W_cont.reshape(_NCONT * _D))
    return out.reshape(_B, 1)


# speedup vs baseline: 1.0635x; 1.0126x over previous
import jax
import jax.numpy as jnp
from jax import lax
from jax.experimental import pallas as pl
from jax.experimental.pallas import tpu as pltpu
from jax.experimental.pallas import tpu_sc as plsc

_B = 16384
_V = 100000
_NDIS = 26
_NCONT = 13
_D = 16

_NCORES = 2
_NSUB = 16
_NW = _NCORES * _NSUB
_BPW = _B // _NW
_NC = 64
_NCHUNK = _BPW // _NC
_NGRP = _NC // 16


def _fm_body(xtr, wfm, wlin, wc, out, xt_v, idxb_v, idx2_v, idx3_v, rows_v,
             lin_v, wc_v, wcb_v, out_v, sem_fm, sem_lin):
    wid = lax.axis_index("s") * _NCORES + lax.axis_index("c")
    pltpu.sync_copy(xtr.at[wid], xt_v)
    pltpu.sync_copy(wc, wc_v)

    lanes = lax.iota(jnp.int32, 16)
    zeros = jnp.zeros((16,), jnp.float32)

    for cf in range(_NCONT):
        row = wc_v[pl.ds(cf * _D, _D)]
        for d in range(_D):
            wcb_v[pl.ds((cf * _D + d) * 16, 16)] = zeros + row[d]

    @pl.loop(0, _BPW // 16)
    def _idx(g):
        off = pl.multiple_of(g * 16, 16)
        for f in range(_NDIS):
            xv = xt_v[pl.ds(f * _BPW + off, 16)]
            iv = xv.astype(jnp.int32)
            idxb_v[pl.ds(f * _BPW + off, 16)] = iv
            idx2_v[pl.ds(f * _BPW + off, 16)] = iv + f * _V

    cp_lin = pltpu.async_copy(wlin.at[idx2_v], lin_v, sem_lin)
    cp_lin.wait()

    @pl.loop(0, _NCHUNK)
    def _chunk(c):
        cbase = c * _NC

        @pl.loop(0, _NGRP)
        def _i3(g):
            off = pl.multiple_of(cbase + g * 16, 16)
            for f in range(_NDIS):
                bx = idxb_v[pl.ds(f * _BPW + off, 16)]
                for d in range(_D):
                    idx3_v[pl.ds((f * _D + d) * _NC + g * 16, 16)] = (
                        bx + (f * _D + d) * _V)

        cp_fm = pltpu.async_copy(wfm.at[idx3_v], rows_v, sem_fm)
        cp_fm.wait()

        @pl.loop(0, _NGRP)
        def _fm(g):
            off = pl.multiple_of(cbase + g * 16, 16)
            goff = g * 16

            linacc = lin_v[pl.ds(off, 16)]
            for f in range(1, _NDIS):
                linacc = linacc + lin_v[pl.ds(f * _BPW + off, 16)]
            for cf in range(_NCONT):
                linacc = linacc + xt_v[pl.ds((_NDIS + cf) * _BPW + off, 16)]

            css = jnp.zeros((16,), jnp.float32)
            cq = jnp.zeros((16,), jnp.float32)
            for h in range(2):
                d0 = h * 8
                s_acc = [jnp.zeros((16,), jnp.float32) for _ in range(8)]
                q_acc = [jnp.zeros((16,), jnp.float32) for _ in range(8)]
                for f in range(_NDIS):
                    xfv = xt_v[pl.ds(f * _BPW + off, 16)]
                    for dd in range(8):
                        rv = rows_v[
                            pl.ds((f * _D + d0 + dd) * _NC + goff, 16)]
                        t = rv * xfv
                        s_acc[dd] = s_acc[dd] + t
                        q_acc[dd] = q_acc[dd] + t * t
                for cf in range(_NCONT):
                    xcv = xt_v[pl.ds((_NDIS + cf) * _BPW + off, 16)]
                    for dd in range(8):
                        t = xcv * wcb_v[pl.ds((cf * _D + d0 + dd) * 16, 16)]
                        s_acc[dd] = s_acc[dd] + t
                        q_acc[dd] = q_acc[dd] + t * t
                for dd in range(8):
                    css = css + s_acc[dd] * s_acc[dd]
                    cq = cq + q_acc[dd]

            z = linacc + 0.5 * (css - cq)
            out_v[pl.ds(off, 16)] = 1.0 / (1.0 + jnp.exp(-z))

    base = pl.multiple_of(wid * _BPW, 8)
    pltpu.sync_copy(out_v, out.at[pl.ds(base, _BPW)])


_fm_call = pl.kernel(
    _fm_body,
    out_type=jax.ShapeDtypeStruct((_B,), jnp.float32),
    mesh=plsc.VectorSubcoreMesh(
        core_axis_name="c", subcore_axis_name="s",
        num_cores=_NCORES, num_subcores=_NSUB),
    scratch_types=[
        pltpu.VMEM(((_NDIS + _NCONT) * _BPW,), jnp.float32),
        pltpu.VMEM((_NDIS * _BPW,), jnp.int32),
        pltpu.VMEM((_NDIS * _BPW,), jnp.int32),
        pltpu.VMEM((_NDIS * _D * _NC,), jnp.int32),
        pltpu.VMEM((_NDIS * _D * _NC,), jnp.float32),
        pltpu.VMEM((_NDIS * _BPW,), jnp.float32),
        pltpu.VMEM((_NCONT * _D,), jnp.float32),
        pltpu.VMEM((_NCONT * _D * 16,), jnp.float32),
        pltpu.VMEM((_BPW,), jnp.float32),
        pltpu.SemaphoreType.DMA,
        pltpu.SemaphoreType.DMA,
    ],
    compiler_params=pltpu.CompilerParams(
        needs_layout_passes=False, use_tc_tiling_on_sc=False),
)


def kernel(X, W_lin, W_fm, W_cont):
    xtr = X.T.reshape(_NDIS + _NCONT, _NW, _BPW).transpose(1, 0, 2)
    xtr = xtr.reshape(_NW, (_NDIS + _NCONT) * _BPW)
    wfm_flat = W_fm.transpose(0, 2, 1).reshape(_NDIS * _D * _V)
    out = _fm_call(xtr, wfm_flat, W_lin.reshape(_NDIS * _V),
                   W_cont.reshape(_NCONT * _D))
    return out.reshape(_B, 1)
